# baseline (speedup 1.0000x reference)
import functools

import jax
import jax.numpy as jnp
from jax import lax
from jax.experimental import pallas as pl
from jax.experimental.pallas import tpu as pltpu
from jax.experimental.pallas import tpu_sc as plsc

N = 10000
E = 320000
D = 128
H = 128
L = 3
F = 8

f32 = jnp.float32

NC = 2
NS = 16
NW = NC * NS
EPW = E // NW
CHUNK = 128
KF = EPW // CHUNK
TAIL = EPW - KF * CHUNK
NBUF = 4
RPT = 632
N_PAD = NS * RPT
CL = 128


@functools.cache
def _mesh():
    return plsc.VectorSubcoreMesh(
        core_axis_name="c", subcore_axis_name="s",
        num_cores=NC, num_subcores=NS)


@functools.cache
def _gather_kernel():
    def body(h_hbm, idxm_hbm, idxt_hbm, out_hbm, idx_v, idxt_v, rows_v,
             gsem, wsem):
        wid = lax.axis_index("s") * NC + lax.axis_index("c")
        pltpu.sync_copy(idxm_hbm.at[wid], idx_v)
        pltpu.sync_copy(idxt_hbm.at[wid], idxt_v)
        base = wid * EPW

        def g_start(j, b):
            pltpu.async_copy(h_hbm.at[idx_v.at[j]], rows_v.at[b], gsem)

        def g_wait(j, b):
            pltpu.make_async_copy(
                h_hbm.at[idx_v.at[j]], rows_v.at[b], gsem).wait()

        def o_start(j, b):
            pltpu.async_copy(
                rows_v.at[b], out_hbm.at[pl.ds(base + j * CHUNK, CHUNK)], wsem)

        def o_wait(j, b):
            pltpu.make_async_copy(
                rows_v.at[b], out_hbm.at[pl.ds(base + j * CHUNK, CHUNK)],
                wsem).wait()

        for j in range(NBUF):
            g_start(j, j)

        def loop(g, _):
            for b in range(NBUF):
                j = g * NBUF + b

                @pl.when(j <= KF - 1)
                def _():
                    g_wait(j, b)
                    o_start(j, b)

                @pl.when(jnp.logical_and(j >= 1, j + NBUF - 1 <= KF - 1))
                def _():
                    o_wait(j - 1, (b + NBUF - 1) % NBUF)
                    g_start(j + NBUF - 1, (b + NBUF - 1) % NBUF)
            return 0

        lax.fori_loop(0, (KF + NBUF - 1) // NBUF, loop, 0)
        for j in range(KF - NBUF, KF):
            o_wait(j, j % NBUF)
        pltpu.async_copy(
            h_hbm.at[idxt_v.at[0]], rows_v.at[0, pl.ds(0, TAIL)], gsem).wait()
        pltpu.sync_copy(rows_v.at[0, pl.ds(0, TAIL)],
                        out_hbm.at[pl.ds(base + KF * CHUNK, TAIL)])

    return functools.partial(
        pl.kernel,
        out_type=jax.ShapeDtypeStruct((E, D), f32),
        mesh=_mesh(),
        name="sc_gather",
        scratch_types=[
            pltpu.VMEM((KF, CHUNK), jnp.int32),
            pltpu.VMEM((1, TAIL), jnp.int32),
            pltpu.VMEM((NBUF, CHUNK, D), f32),
            pltpu.SemaphoreType.DMA,
            pltpu.SemaphoreType.DMA,
        ],
    )(body)


@functools.cache
def _scatter_kernel():
    def body(msg_hbm, idxm_hbm, idxt_hbm, zeros_hbm, s_out,
             idx_v, idxt_v, msg_v, acc, lsem):
        cid = lax.axis_index("c")
        sid = lax.axis_index("s")
        wid = sid * NC + cid
        pltpu.sync_copy(zeros_hbm.at[pl.ds(sid * RPT, RPT)],
                        acc.at[pl.ds(sid * RPT, RPT)])
        pltpu.sync_copy(idxm_hbm.at[wid], idx_v)
        pltpu.sync_copy(idxt_hbm.at[wid], idxt_v)
        plsc.subcore_barrier()

        base = wid * EPW

        def l_start(j, b):
            pltpu.async_copy(
                msg_hbm.at[pl.ds(base + j * CHUNK, CHUNK)], msg_v.at[b], lsem)

        def l_wait(j, b):
            pltpu.make_async_copy(
                msg_hbm.at[pl.ds(base + j * CHUNK, CHUNK)], msg_v.at[b],
                lsem).wait()

        l_start(0, 0)

        def loop(g, _):
            for b in range(2):
                j = g * 2 + b

                @pl.when(j + 1 <= KF - 1)
                def _():
                    l_start(j + 1, 1 - b)

                @pl.when(j <= KF - 1)
                def _():
                    l_wait(j, b)
                    pltpu.sync_copy(msg_v.at[b], acc.at[idx_v.at[j]], add=True)
            return 0

        lax.fori_loop(0, (KF + 1) // 2, loop, 0)
        pltpu.sync_copy(msg_hbm.at[pl.ds(base + KF * CHUNK, TAIL)],
                        msg_v.at[0, pl.ds(0, TAIL)])
        pltpu.sync_copy(msg_v.at[0, pl.ds(0, TAIL)],
                        acc.at[idxt_v.at[0]], add=True)
        plsc.subcore_barrier()
        pltpu.sync_copy(acc.at[pl.ds(sid * RPT, RPT)],
                        s_out.at[pl.ds(cid * N_PAD + sid * RPT, RPT)])

    return functools.partial(
        pl.kernel,
        out_type=jax.ShapeDtypeStruct((NC * N_PAD, D), f32),
        mesh=_mesh(),
        name="sc_scatter",
        scratch_types=[
            pltpu.VMEM((KF, CHUNK), jnp.int32),
            pltpu.VMEM((1, TAIL), jnp.int32),
            pltpu.VMEM((2, CHUNK, D), f32),
            pltpu.VMEM_SHARED((N_PAD, D), f32),
            pltpu.SemaphoreType.DMA,
        ],
    )(body)


@functools.cache
def _cnt_kernel():
    def body(idxm_hbm, idxt_hbm, czeros_hbm, ones_hbm, cnt_out,
             idx_v, idxt_v, ones_v, cacc):
        cid = lax.axis_index("c")
        sid = lax.axis_index("s")
        wid = sid * NC + cid
        pltpu.sync_copy(czeros_hbm.at[pl.ds(sid * RPT, RPT)],
                        cacc.at[pl.ds(sid * RPT, RPT)])
        pltpu.sync_copy(ones_hbm, ones_v)
        pltpu.sync_copy(idxm_hbm.at[wid], idx_v)
        pltpu.sync_copy(idxt_hbm.at[wid], idxt_v)
        plsc.subcore_barrier()

        def loop(j, _):
            pltpu.sync_copy(ones_v, cacc.at[idx_v.at[j]], add=True)
            return 0

        lax.fori_loop(0, KF, loop, 0)
        pltpu.sync_copy(ones_v.at[pl.ds(0, TAIL)],
                        cacc.at[idxt_v.at[0]], add=True)
        plsc.subcore_barrier()
        pltpu.sync_copy(cacc.at[pl.ds(sid * RPT, RPT)],
                        cnt_out.at[pl.ds(cid * N_PAD + sid * RPT, RPT)])

    return functools.partial(
        pl.kernel,
        out_type=jax.ShapeDtypeStruct((NC * N_PAD, CL), f32),
        mesh=_mesh(),
        name="sc_cnt",
        scratch_types=[
            pltpu.VMEM((KF, CHUNK), jnp.int32),
            pltpu.VMEM((1, TAIL), jnp.int32),
            pltpu.VMEM((CHUNK, CL), f32),
            pltpu.VMEM_SHARED((N_PAD, CL), f32),
        ],
    )(body)


BE = 10000


def _msg_body(hs, ef, e, w1h, w1f, w1e, b1, w2, b2, msg_o):
    t = jnp.dot(hs[...], w1h[...], preferred_element_type=f32)
    t += jnp.dot(ef[...], w1f[...], preferred_element_type=f32)
    t += jnp.dot(e[...].astype(f32), w1e[...], preferred_element_type=f32)
    t = jnp.maximum(t + b1[...], 0.0)
    msg_o[...] = jnp.dot(t, w2[...], preferred_element_type=f32) + b2[...]


def _msg_mlp(hs, ef, e, w1h, w1f, w1e, b1, w2, b2):
    blk = lambda r, c: pl.BlockSpec((r, c), lambda i: (i, 0))
    full = lambda r, c: pl.BlockSpec((r, c), lambda i: (0, 0))
    return pl.pallas_call(
        _msg_body,
        grid=(E // BE,),
        in_specs=[
            blk(BE, D), blk(BE, F), blk(BE, D),
            full(D, H), full(F, H), full(D, H), full(1, H),
            full(H, H), full(1, H),
        ],
        out_specs=blk(BE, D),
        out_shape=jax.ShapeDtypeStruct((E, D), f32),
        compiler_params=pltpu.CompilerParams(
            dimension_semantics=("arbitrary",)),
    )(hs, ef, e, w1h, w1f, w1e, b1, w2, b2)


def _e_body(msg, we1, eb1, we2, eb2, e_o):
    u = jnp.maximum(
        jnp.dot(msg[...], we1[...], preferred_element_type=f32) + eb1[...],
        0.0)
    res = jnp.dot(u, we2[...], preferred_element_type=f32) + eb2[...]
    e_o[...] = res.astype(e_o.dtype)


def _e_mlp(msg, we1, eb1, we2, eb2, out_dtype):
    blk = lambda r, c: pl.BlockSpec((r, c), lambda i: (i, 0))
    full = lambda r, c: pl.BlockSpec((r, c), lambda i: (0, 0))
    return pl.pallas_call(
        _e_body,
        grid=(E // BE,),
        in_specs=[
            blk(BE, D),
            full(H, H), full(1, H), full(H, D), full(1, D),
        ],
        out_specs=blk(BE, D),
        out_shape=jax.ShapeDtypeStruct((E, D), out_dtype),
        compiler_params=pltpu.CompilerParams(
            dimension_semantics=("arbitrary",)),
    )(msg, we1, eb1, we2, eb2)


BN = 1000


def _node_body(s0, s1, c0, c1, h, w1a, w1h, b1, w2, b2, out):
    cnt = c0[...][:, :1] + c1[...][:, :1]
    agg = (s0[...] + s1[...]) / jnp.maximum(cnt, 1.0)
    t = jnp.dot(agg, w1a[...], preferred_element_type=f32)
    t += jnp.dot(h[...], w1h[...], preferred_element_type=f32)
    t = jnp.maximum(t + b1[...], 0.0)
    out[...] = jnp.dot(t, w2[...], preferred_element_type=f32) + b2[...]


def _node_mlp(s0, s1, c0, c1, h, w1a, w1h, b1, w2, b2):
    blk = lambda r, c: pl.BlockSpec((r, c), lambda i: (i, 0))
    full = lambda r, c: pl.BlockSpec((r, c), lambda i: (0, 0))
    return pl.pallas_call(
        _node_body,
        grid=(N // BN,),
        in_specs=[
            blk(BN, D), blk(BN, D), blk(BN, CL), blk(BN, CL), blk(BN, D),
            full(H, H), full(D, H), full(1, H), full(H, D), full(1, D),
        ],
        out_specs=blk(BN, D),
        out_shape=jax.ShapeDtypeStruct((N, D), f32),
        compiler_params=pltpu.CompilerParams(
            dimension_semantics=("arbitrary",)),
    )(s0, s1, c0, c1, h, w1a, w1h, b1, w2, b2)


def kernel(x, edge_index, edge_feat, edge_emb0,
           Wm1, bm1, Wm2, bm2, Wn1, bn1, Wn2, bn2, We1, be1, We2, be2):
    def split_idx(v):
        v = v.reshape(NW, EPW)
        return (v[:, :KF * CHUNK].reshape(NW, KF, CHUNK),
                v[:, KF * CHUNK:].reshape(NW, 1, TAIL))

    srcm, srct = split_idx(edge_index[0])
    dstm, dstt = split_idx(edge_index[1])
    zeros_nd = jnp.zeros((N_PAD, D), f32)
    czeros = jnp.zeros((N_PAD, CL), f32)
    ones_c = jnp.ones((CHUNK, CL), f32)

    h = x
    e = edge_emb0
    cnt = _cnt_kernel()(dstm, dstt, czeros, ones_c)
    for l in range(L):
        hs = _gather_kernel()(h, srcm, srct)
        msg = _msg_mlp(
            hs, edge_feat, e,
            Wm1[l, :D], Wm1[l, D:D + F], Wm1[l, D + F:], bm1[l:l + 1],
            Wm2[l], bm2[l:l + 1])
        s = _scatter_kernel()(msg, dstm, dstt, zeros_nd)
        e_dtype = f32 if l == L - 1 else jnp.bfloat16
        e = _e_mlp(msg, We1[l], be1[l:l + 1], We2[l], be2[l:l + 1], e_dtype)
        h = _node_mlp(
            s[:N], s[N_PAD:N_PAD + N], cnt[:N], cnt[N_PAD:N_PAD + N], h,
            Wn1[l, :H], Wn1[l, H:], bn1[l:l + 1], Wn2[l], bn2[l:l + 1])
    return (h, e)


# speedup vs baseline: 1.0045x; 1.0045x over previous
import functools

import jax
import jax.numpy as jnp
from jax import lax
from jax.experimental import pallas as pl
from jax.experimental.pallas import tpu as pltpu
from jax.experimental.pallas import tpu_sc as plsc

N = 10000
E = 320000
D = 128
H = 128
L = 3
F = 8

f32 = jnp.float32

NC = 2
NS = 16
NW = NC * NS
EPW = E // NW
CHUNK = 128
KF = EPW // CHUNK
TAIL = EPW - KF * CHUNK
NBUF = 4
RPT = 632
N_PAD = NS * RPT
CL = 128


@functools.cache
def _mesh():
    return plsc.VectorSubcoreMesh(
        core_axis_name="c", subcore_axis_name="s",
        num_cores=NC, num_subcores=NS)


@functools.cache
def _gather_kernel():
    def body(h_hbm, idxm_hbm, idxt_hbm, out_hbm, idx_v, idxt_v, rows_v,
             gsem, wsem):
        wid = lax.axis_index("s") * NC + lax.axis_index("c")
        pltpu.sync_copy(idxm_hbm.at[wid], idx_v)
        pltpu.sync_copy(idxt_hbm.at[wid], idxt_v)
        base = wid * EPW

        def g_start(j, b):
            pltpu.async_copy(h_hbm.at[idx_v.at[j]], rows_v.at[b], gsem)

        def g_wait(j, b):
            pltpu.make_async_copy(
                h_hbm.at[idx_v.at[j]], rows_v.at[b], gsem).wait()

        def o_start(j, b):
            pltpu.async_copy(
                rows_v.at[b], out_hbm.at[pl.ds(base + j * CHUNK, CHUNK)], wsem)

        def o_wait(j, b):
            pltpu.make_async_copy(
                rows_v.at[b], out_hbm.at[pl.ds(base + j * CHUNK, CHUNK)],
                wsem).wait()

        for j in range(NBUF):
            g_start(j, j)

        def loop(g, _):
            for b in range(NBUF):
                j = g * NBUF + b

                @pl.when(j <= KF - 1)
                def _():
                    g_wait(j, b)
                    o_start(j, b)

                @pl.when(jnp.logical_and(j >= 1, j + NBUF - 1 <= KF - 1))
                def _():
                    o_wait(j - 1, (b + NBUF - 1) % NBUF)
                    g_start(j + NBUF - 1, (b + NBUF - 1) % NBUF)
            return 0

        lax.fori_loop(0, (KF + NBUF - 1) // NBUF, loop, 0)
        for j in range(KF - NBUF, KF):
            o_wait(j, j % NBUF)
        pltpu.async_copy(
            h_hbm.at[idxt_v.at[0]], rows_v.at[0, pl.ds(0, TAIL)], gsem).wait()
        pltpu.sync_copy(rows_v.at[0, pl.ds(0, TAIL)],
                        out_hbm.at[pl.ds(base + KF * CHUNK, TAIL)])

    return functools.partial(
        pl.kernel,
        out_type=jax.ShapeDtypeStruct((E, D), f32),
        mesh=_mesh(),
        name="sc_gather",
        scratch_types=[
            pltpu.VMEM((KF, CHUNK), jnp.int32),
            pltpu.VMEM((1, TAIL), jnp.int32),
            pltpu.VMEM((NBUF, CHUNK, D), f32),
            pltpu.SemaphoreType.DMA,
            pltpu.SemaphoreType.DMA,
        ],
    )(body)


@functools.cache
def _scatter_kernel():
    def body(msg_hbm, idxm_hbm, idxt_hbm, zeros_hbm, s_out,
             idx_v, idxt_v, msg_v, acc, lsem):
        cid = lax.axis_index("c")
        sid = lax.axis_index("s")
        wid = sid * NC + cid
        pltpu.sync_copy(zeros_hbm.at[pl.ds(sid * RPT, RPT)],
                        acc.at[pl.ds(sid * RPT, RPT)])
        pltpu.sync_copy(idxm_hbm.at[wid], idx_v)
        pltpu.sync_copy(idxt_hbm.at[wid], idxt_v)
        plsc.subcore_barrier()

        base = wid * EPW

        def l_start(j, b):
            pltpu.async_copy(
                msg_hbm.at[pl.ds(base + j * CHUNK, CHUNK)], msg_v.at[b], lsem)

        def l_wait(j, b):
            pltpu.make_async_copy(
                msg_hbm.at[pl.ds(base + j * CHUNK, CHUNK)], msg_v.at[b],
                lsem).wait()

        l_start(0, 0)

        def loop(g, _):
            for b in range(2):
                j = g * 2 + b

                @pl.when(j + 1 <= KF - 1)
                def _():
                    l_start(j + 1, 1 - b)

                @pl.when(j <= KF - 1)
                def _():
                    l_wait(j, b)
                    pltpu.sync_copy(msg_v.at[b], acc.at[idx_v.at[j]], add=True)
            return 0

        lax.fori_loop(0, (KF + 1) // 2, loop, 0)
        pltpu.sync_copy(msg_hbm.at[pl.ds(base + KF * CHUNK, TAIL)],
                        msg_v.at[0, pl.ds(0, TAIL)])
        pltpu.sync_copy(msg_v.at[0, pl.ds(0, TAIL)],
                        acc.at[idxt_v.at[0]], add=True)
        plsc.subcore_barrier()
        pltpu.sync_copy(acc.at[pl.ds(sid * RPT, RPT)],
                        s_out.at[pl.ds(cid * N_PAD + sid * RPT, RPT)])

    return functools.partial(
        pl.kernel,
        out_type=jax.ShapeDtypeStruct((NC * N_PAD, D), f32),
        mesh=_mesh(),
        name="sc_scatter",
        scratch_types=[
            pltpu.VMEM((KF, CHUNK), jnp.int32),
            pltpu.VMEM((1, TAIL), jnp.int32),
            pltpu.VMEM((2, CHUNK, D), f32),
            pltpu.VMEM_SHARED((N_PAD, D), f32),
            pltpu.SemaphoreType.DMA,
        ],
    )(body)


@functools.cache
def _cnt_kernel():
    def body(idxm_hbm, idxt_hbm, czeros_hbm, ones_hbm, cnt_out,
             idx_v, idxt_v, ones_v, cacc):
        cid = lax.axis_index("c")
        sid = lax.axis_index("s")
        wid = sid * NC + cid
        pltpu.sync_copy(czeros_hbm.at[pl.ds(sid * RPT, RPT)],
                        cacc.at[pl.ds(sid * RPT, RPT)])
        pltpu.sync_copy(ones_hbm, ones_v)
        pltpu.sync_copy(idxm_hbm.at[wid], idx_v)
        pltpu.sync_copy(idxt_hbm.at[wid], idxt_v)
        plsc.subcore_barrier()

        def loop(j, _):
            pltpu.sync_copy(ones_v, cacc.at[idx_v.at[j]], add=True)
            return 0

        lax.fori_loop(0, KF, loop, 0)
        pltpu.sync_copy(ones_v.at[pl.ds(0, TAIL)],
                        cacc.at[idxt_v.at[0]], add=True)
        plsc.subcore_barrier()
        pltpu.sync_copy(cacc.at[pl.ds(sid * RPT, RPT)],
                        cnt_out.at[pl.ds(cid * N_PAD + sid * RPT, RPT)])

    return functools.partial(
        pl.kernel,
        out_type=jax.ShapeDtypeStruct((NC * N_PAD, CL), f32),
        mesh=_mesh(),
        name="sc_cnt",
        scratch_types=[
            pltpu.VMEM((KF, CHUNK), jnp.int32),
            pltpu.VMEM((1, TAIL), jnp.int32),
            pltpu.VMEM((CHUNK, CL), f32),
            pltpu.VMEM_SHARED((N_PAD, CL), f32),
        ],
    )(body)


BE = 8000


def _msg_body(hs, ef, e, w1h, w1f, w1e, b1, w2, b2, msg_o):
    t = jnp.dot(hs[...], w1h[...], preferred_element_type=f32)
    t += jnp.dot(ef[...], w1f[...], preferred_element_type=f32)
    t += jnp.dot(e[...].astype(f32), w1e[...], preferred_element_type=f32)
    t = jnp.maximum(t + b1[...], 0.0)
    msg_o[...] = jnp.dot(t, w2[...], preferred_element_type=f32) + b2[...]


def _msg_mlp(hs, ef, e, w1h, w1f, w1e, b1, w2, b2):
    blk = lambda r, c: pl.BlockSpec((r, c), lambda i: (i, 0))
    full = lambda r, c: pl.BlockSpec((r, c), lambda i: (0, 0))
    return pl.pallas_call(
        _msg_body,
        grid=(E // BE,),
        in_specs=[
            blk(BE, D), blk(BE, F), blk(BE, D),
            full(D, H), full(F, H), full(D, H), full(1, H),
            full(H, H), full(1, H),
        ],
        out_specs=blk(BE, D),
        out_shape=jax.ShapeDtypeStruct((E, D), f32),
        compiler_params=pltpu.CompilerParams(
            dimension_semantics=("arbitrary",)),
    )(hs, ef, e, w1h, w1f, w1e, b1, w2, b2)


def _e_body(msg, we1, eb1, we2, eb2, e_o):
    u = jnp.maximum(
        jnp.dot(msg[...], we1[...], preferred_element_type=f32) + eb1[...],
        0.0)
    res = jnp.dot(u, we2[...], preferred_element_type=f32) + eb2[...]
    e_o[...] = res.astype(e_o.dtype)


def _e_mlp(msg, we1, eb1, we2, eb2, out_dtype):
    blk = lambda r, c: pl.BlockSpec((r, c), lambda i: (i, 0))
    full = lambda r, c: pl.BlockSpec((r, c), lambda i: (0, 0))
    return pl.pallas_call(
        _e_body,
        grid=(E // BE,),
        in_specs=[
            blk(BE, D),
            full(H, H), full(1, H), full(H, D), full(1, D),
        ],
        out_specs=blk(BE, D),
        out_shape=jax.ShapeDtypeStruct((E, D), out_dtype),
        compiler_params=pltpu.CompilerParams(
            dimension_semantics=("arbitrary",)),
    )(msg, we1, eb1, we2, eb2)


BN = 1000


def _node_body(s0, s1, c0, c1, h, w1a, w1h, b1, w2, b2, out):
    cnt = c0[...][:, :1] + c1[...][:, :1]
    agg = (s0[...] + s1[...]) / jnp.maximum(cnt, 1.0)
    t = jnp.dot(agg, w1a[...], preferred_element_type=f32)
    t += jnp.dot(h[...], w1h[...], preferred_element_type=f32)
    t = jnp.maximum(t + b1[...], 0.0)
    out[...] = jnp.dot(t, w2[...], preferred_element_type=f32) + b2[...]


def _node_mlp(s0, s1, c0, c1, h, w1a, w1h, b1, w2, b2):
    blk = lambda r, c: pl.BlockSpec((r, c), lambda i: (i, 0))
    full = lambda r, c: pl.BlockSpec((r, c), lambda i: (0, 0))
    return pl.pallas_call(
        _node_body,
        grid=(N // BN,),
        in_specs=[
            blk(BN, D), blk(BN, D), blk(BN, CL), blk(BN, CL), blk(BN, D),
            full(H, H), full(D, H), full(1, H), full(H, D), full(1, D),
        ],
        out_specs=blk(BN, D),
        out_shape=jax.ShapeDtypeStruct((N, D), f32),
        compiler_params=pltpu.CompilerParams(
            dimension_semantics=("arbitrary",)),
    )(s0, s1, c0, c1, h, w1a, w1h, b1, w2, b2)


def kernel(x, edge_index, edge_feat, edge_emb0,
           Wm1, bm1, Wm2, bm2, Wn1, bn1, Wn2, bn2, We1, be1, We2, be2):
    def split_idx(v):
        v = v.reshape(NW, EPW)
        return (v[:, :KF * CHUNK].reshape(NW, KF, CHUNK),
                v[:, KF * CHUNK:].reshape(NW, 1, TAIL))

    srcm, srct = split_idx(edge_index[0])
    dstm, dstt = split_idx(edge_index[1])
    zeros_nd = jnp.zeros((N_PAD, D), f32)
    czeros = jnp.zeros((N_PAD, CL), f32)
    ones_c = jnp.ones((CHUNK, CL), f32)

    h = x
    e = edge_emb0
    cnt = _cnt_kernel()(dstm, dstt, czeros, ones_c)
    for l in range(L):
        hs = _gather_kernel()(h, srcm, srct)
        msg = _msg_mlp(
            hs, edge_feat, e,
            Wm1[l, :D], Wm1[l, D:D + F], Wm1[l, D + F:], bm1[l:l + 1],
            Wm2[l], bm2[l:l + 1])
        s = _scatter_kernel()(msg, dstm, dstt, zeros_nd)
        e_dtype = f32 if l == L - 1 else jnp.bfloat16
        e = _e_mlp(msg, We1[l], be1[l:l + 1], We2[l], be2[l:l + 1], e_dtype)
        h = _node_mlp(
            s[:N], s[N_PAD:N_PAD + N], cnt[:N], cnt[N_PAD:N_PAD + N], h,
            Wn1[l, :H], Wn1[l, H:], bn1[l:l + 1], Wn2[l], bn2[l:l + 1])
    return (h, e)
